# baseline (speedup 1.0000x reference)
import functools
import jax
import jax.numpy as jnp
from jax import lax
from jax.experimental import pallas as pl
from jax.experimental.pallas import tpu as pltpu
from jax.experimental.pallas import tpu_sc as plsc

_HI = lax.Precision.HIGHEST


def _sqdist(qp_ref, rt_ref):
    q = qp_ref[...]
    rt = rt_ref[...]
    q2 = (q[:, 0:1] * q[:, 0:1] + q[:, 2:3] * q[:, 2:3]) + q[:, 1:2] * q[:, 1:2]
    r2 = (rt[0:1, :] * rt[0:1, :] + rt[2:3, :] * rt[2:3, :]) + rt[1:2, :] * rt[1:2, :]
    dot = lax.dot(q.astype(jnp.bfloat16), rt.astype(jnp.bfloat16),
                  preferred_element_type=jnp.float32)
    return q2 + r2 - 2.0 * dot


def _knn_topk(qp_ref, rt_ref, k, m_total):
    qb = qp_ref.shape[0]
    d = _sqdist(qp_ref, rt_ref)
    nseg = m_total // 128
    d3 = d.reshape(qb, nseg, 128)
    inf = jnp.float32(jnp.inf)
    bigi = jnp.int32(m_total)
    li3 = lax.broadcasted_iota(jnp.int32, (qb, nseg, 128), 2)
    smin = jnp.min(d3, axis=2)
    sl = jnp.min(jnp.where(d3 == smin[:, :, None], li3, 128), axis=2)
    seg_iota = lax.broadcasted_iota(jnp.int32, (qb, nseg), 1)
    gmin = seg_iota * 128 + sl
    ksel = lax.broadcasted_iota(jnp.int32, (qb, k), 1)
    sv = smin
    seg_sel = jnp.zeros((qb, k), jnp.int32)
    for t in range(k):
        m = jnp.min(sv, axis=1, keepdims=True)
        c = jnp.where(sv == m, gmin, bigi)
        a = jnp.min(c, axis=1, keepdims=True)
        seg_sel = jnp.where(ksel == t, a // 128, seg_sel)
        sv = jnp.where(c == a, inf, sv)
    comp = []
    for t in range(k):
        st = seg_sel[:, t:t + 1]
        pen = jnp.where(seg_iota == st, 0.0, inf)
        comp.append(jnp.min(d3 + pen[:, :, None], axis=1)[:, None, :])
    vals3 = jnp.concatenate(comp, axis=1)
    li3k = lax.broadcasted_iota(jnp.int32, (qb, k, 128), 2)
    gidx3 = seg_sel[:, :, None] * 128 + li3k
    acc = jnp.zeros((qb, k), jnp.int32)
    dacc = jnp.zeros((qb, k), jnp.float32)
    for t in range(k):
        m = jnp.min(jnp.min(vals3, axis=2, keepdims=True), axis=1, keepdims=True)
        cnd = jnp.where(vals3 == m, gidx3, bigi)
        a = jnp.min(jnp.min(cnd, axis=2, keepdims=True), axis=1, keepdims=True)
        acc = jnp.where(ksel == t, a[:, :, 0], acc)
        dacc = jnp.where(ksel == t, m[:, :, 0], dacc)
        vals3 = jnp.where(cnd == a, inf, vals3)
    return acc, dacc


def _knn16_body(qp_ref, rt_ref, idx_ref, *, m_total):
    acc, _ = _knn_topk(qp_ref, rt_ref, 16, m_total)
    idx_ref[...] = acc


def _run_knn16(qp, rt, qb):
    q_n = qp.shape[0]
    m = rt.shape[1]
    return pl.pallas_call(
        functools.partial(_knn16_body, m_total=m),
        grid=(q_n // qb,),
        in_specs=[pl.BlockSpec((qb, 8), lambda i: (i, 0)),
                  pl.BlockSpec((8, m), lambda i: (0, 0))],
        out_specs=pl.BlockSpec((qb, 16), lambda i: (i, 0)),
        out_shape=jax.ShapeDtypeStruct((q_n, 16), jnp.int32),
        compiler_params=pltpu.CompilerParams(
            dimension_semantics=("parallel",)),
    )(qp, rt)


def _knn3_body(qp_ref, rt_ref, idx_ref, w_ref, *, m_total):
    acc, dacc = _knn_topk(qp_ref, rt_ref, 3, m_total)
    wacc = 1.0 / (dacc + 1e-8)
    wsum = (wacc[:, 0:1] + wacc[:, 2:3]) + wacc[:, 1:2]
    idx_ref[...] = acc
    w_ref[...] = wacc / wsum


def _run_knn3(qp, rt, qb):
    q_n = qp.shape[0]
    m = rt.shape[1]
    return pl.pallas_call(
        functools.partial(_knn3_body, m_total=m),
        grid=(q_n // qb,),
        in_specs=[pl.BlockSpec((qb, 8), lambda i: (i, 0)),
                  pl.BlockSpec((8, m), lambda i: (0, 0))],
        out_specs=[pl.BlockSpec((qb, 3), lambda i: (i, 0)),
                   pl.BlockSpec((qb, 3), lambda i: (i, 0))],
        out_shape=[jax.ShapeDtypeStruct((q_n, 3), jnp.int32),
                   jax.ShapeDtypeStruct((q_n, 3), jnp.float32)],
        compiler_params=pltpu.CompilerParams(
            dimension_semantics=("parallel",)),
    )(qp, rt)


_NC = 2
_NS = 16
_NW = _NC * _NS


def _sc_gather(table, idx, chunk):
    b_n = idx.shape[0]
    d = table.shape[1]
    b_per_w = b_n // _NW
    nch = b_per_w // chunk
    assert b_per_w % chunk == 0 and b_n % (8 * _NW) == 0 and d % 128 == 0
    mesh = plsc.VectorSubcoreMesh(core_axis_name="c", subcore_axis_name="s")

    @functools.partial(
        pl.kernel, mesh=mesh,
        out_type=jax.ShapeDtypeStruct((b_n, d), jnp.float32),
        scratch_types=[pltpu.VMEM((b_per_w,), jnp.int32),
                       pltpu.VMEM((chunk, d), jnp.float32),
                       pltpu.SemaphoreType.DMA],
    )
    def k(table_hbm, idx_hbm, out_hbm, idx_v, buf, sem):
        wid = lax.axis_index("s") * _NC + lax.axis_index("c")
        base = wid * b_per_w
        pltpu.sync_copy(idx_hbm.at[pl.ds(base, b_per_w)], idx_v)
        for c in range(nch):
            pltpu.async_copy(
                table_hbm.at[idx_v.at[pl.ds(c * chunk, chunk)]], buf, sem
            ).wait()
            pltpu.sync_copy(buf, out_hbm.at[pl.ds(base + c * chunk, chunk)])

    return k(table, idx)


def _bdot(x, w_ref):
    return lax.dot(x.astype(jnp.bfloat16), w_ref[...],
                   preferred_element_type=jnp.float32)


def _sa_mlp_body(g_ref, cor_ref, w0_ref, b0_ref, w1_ref, b1_ref,
                 w2_ref, b2_ref, out_ref, *, qb):
    g = g_ref[...]
    dpad = g.shape[1]
    cor = cor_ref[...]
    cor16 = jnp.broadcast_to(cor[:, None, :], (qb, 16, dpad)).reshape(qb * 16, dpad)
    h0 = jnp.maximum(_bdot(g - cor16, w0_ref) + b0_ref[...], 0.0)
    h1 = jnp.maximum(_bdot(h0, w1_ref) + b1_ref[...], 0.0)
    h2 = jnp.maximum(_bdot(h1, w2_ref) + b2_ref[...], 0.0)
    out_ref[...] = jnp.max(h2.reshape(qb, 16, h2.shape[1]), axis=1)


def _sa_mlp(g, cor, w0p, b0, w1, b1, w2, b2, qb):
    q_n = cor.shape[0]
    dpad = g.shape[1]
    c_out = w2.shape[1]
    c1 = w0p.shape[1]
    c2 = w1.shape[1]
    full = lambda shape: pl.BlockSpec(shape, lambda i: tuple(0 for _ in shape))
    return pl.pallas_call(
        functools.partial(_sa_mlp_body, qb=qb),
        grid=(q_n // qb,),
        in_specs=[pl.BlockSpec((qb * 16, dpad), lambda i: (i, 0)),
                  pl.BlockSpec((qb, dpad), lambda i: (i, 0)),
                  full((dpad, c1)), full((1, c1)),
                  full((c1, c2)), full((1, c2)),
                  full((c2, c_out)), full((1, c_out))],
        out_specs=pl.BlockSpec((qb, c_out), lambda i: (i, 0)),
        out_shape=jax.ShapeDtypeStruct((q_n, c_out), jnp.float32),
        compiler_params=pltpu.CompilerParams(
            dimension_semantics=("parallel",)),
    )(g, cor, w0p, b0, w1, b1, w2, b2)


def _fp_body(gr_ref, w0c_ref, w1c_ref, w2c_ref, skip_ref, wi_ref, ws_ref,
             b0_ref, w1_ref, b1_ref, out_ref):
    x = gr_ref[...]
    c = wi_ref.shape[0]
    interp = (x[:, :c] * w0c_ref[...] + x[:, c:2 * c] * w1c_ref[...]
              + x[:, 2 * c:] * w2c_ref[...])
    h0 = (_bdot(interp, wi_ref) + _bdot(skip_ref[...], ws_ref) + b0_ref[...])
    h0 = jnp.maximum(h0, 0.0)
    h1 = jnp.maximum(_bdot(h0, w1_ref) + b1_ref[...], 0.0)
    out_ref[...] = h1


def _fp_mlp(gr, w3, skip, wi, ws, b0, w1, b1, qb):
    q_n = gr.shape[0]
    c3 = gr.shape[1]
    c = wi.shape[0]
    cs = ws.shape[0]
    c1 = wi.shape[1]
    c2 = w1.shape[1]
    w0c = w3[:, 0:1]
    w1c = w3[:, 1:2]
    w2c = w3[:, 2:3]
    full = lambda shape: pl.BlockSpec(shape, lambda i: tuple(0 for _ in shape))
    return pl.pallas_call(
        _fp_body,
        grid=(q_n // qb,),
        in_specs=[pl.BlockSpec((qb, c3), lambda i: (i, 0)),
                  pl.BlockSpec((qb, 1), lambda i: (i, 0)),
                  pl.BlockSpec((qb, 1), lambda i: (i, 0)),
                  pl.BlockSpec((qb, 1), lambda i: (i, 0)),
                  pl.BlockSpec((qb, cs), lambda i: (i, 0)),
                  full((c, c1)), full((cs, c1)), full((1, c1)),
                  full((c1, c2)), full((1, c2))],
        out_specs=pl.BlockSpec((qb, c2), lambda i: (i, 0)),
        out_shape=jax.ShapeDtypeStruct((q_n, c2), jnp.float32),
        compiler_params=pltpu.CompilerParams(
            dimension_semantics=("parallel",)),
    )(gr, w0c, w1c, w2c, skip, wi, ws, b0, w1, b1)


def _pad_cols(x, d):
    return jnp.pad(x, ((0, 0), (0, d - x.shape[1])))


def kernel(point_bxyz, point_feat, sa1_W0, sa1_b0, sa1_W1, sa1_b1, sa1_W2,
           sa1_b2, sa2_W0, sa2_b0, sa2_W1, sa2_b1, sa2_W2, sa2_b2, fp1_W0,
           fp1_b0, fp1_W1, fp1_b1, fp2_W0, fp2_b0, fp2_W1, fp2_b1):
    xyz0 = point_bxyz[:, 1:4]
    qp0 = _pad_cols(xyz0, 8)
    qp1 = qp0[::4]
    qp2 = qp1[::4]
    rt0 = qp0.T
    rt1 = qp1.T
    rt2 = qp2.T

    row = lambda b: b.reshape(1, -1)
    bf = lambda w: w.astype(jnp.bfloat16)
    xyz1 = xyz0[::4]
    xyz2 = xyz1[::4]

    nn1 = _run_knn16(qp1, rt0, qb=64)
    table1 = _pad_cols(jnp.concatenate([point_feat, xyz0], axis=1), 128)
    g1 = _sc_gather(table1, nn1.reshape(-1), chunk=512)
    w0p = _pad_cols(sa1_W0.T, 128).T
    cor1 = _pad_cols(jnp.concatenate([jnp.zeros((4096, 64), jnp.float32),
                                      xyz1], axis=1), 128)
    f1 = _sa_mlp(g1, cor1, bf(w0p), row(sa1_b0), bf(sa1_W1), row(sa1_b1),
                 bf(sa1_W2), row(sa1_b2), qb=512)

    nn2 = _run_knn16(qp2, rt1, qb=128)
    table2 = _pad_cols(jnp.concatenate([f1, xyz1], axis=1), 256)
    g2 = _sc_gather(table2, nn2.reshape(-1), chunk=256)
    w0p2 = _pad_cols(sa2_W0.T, 256).T
    cor2 = _pad_cols(jnp.concatenate([jnp.zeros((1024, 128), jnp.float32),
                                      xyz2], axis=1), 256)
    f2 = _sa_mlp(g2, cor2, bf(w0p2), row(sa2_b0), bf(sa2_W1), row(sa2_b1),
                 bf(sa2_W2), row(sa2_b2), qb=512)

    nn3a, w3a = _run_knn3(qp1, rt2, qb=256)
    gf1 = _sc_gather(f2, nn3a.reshape(-1), chunk=384)
    f1p = _fp_mlp(gf1.reshape(4096, 768), w3a, f1, bf(fp1_W0[:256]),
                  bf(fp1_W0[256:]), row(fp1_b0), bf(fp1_W1), row(fp1_b1),
                  qb=512)

    nn3b, w3b = _run_knn3(qp0, rt1, qb=256)
    gf2 = _sc_gather(f1p, nn3b.reshape(-1), chunk=384)
    out = _fp_mlp(gf2.reshape(16384, 768), w3b, point_feat, bf(fp2_W0[:256]),
                  bf(fp2_W0[256:]), row(fp2_b0), bf(fp2_W1), row(fp2_b1),
                  qb=1024)
    return out


# speedup vs baseline: 3.6228x; 3.6228x over previous
import functools
import jax
import jax.numpy as jnp
from jax import lax
from jax.experimental import pallas as pl
from jax.experimental.pallas import tpu as pltpu
from jax.experimental.pallas import tpu_sc as plsc

_HI = lax.Precision.HIGHEST


def _sqdist(qp_ref, rt_ref):
    q = qp_ref[...]
    rt = rt_ref[...]
    q2 = (q[:, 0:1] * q[:, 0:1] + q[:, 2:3] * q[:, 2:3]) + q[:, 1:2] * q[:, 1:2]
    r2 = (rt[0:1, :] * rt[0:1, :] + rt[2:3, :] * rt[2:3, :]) + rt[1:2, :] * rt[1:2, :]
    dot = lax.dot(q.astype(jnp.bfloat16), rt.astype(jnp.bfloat16),
                  preferred_element_type=jnp.float32)
    return q2 + r2 - 2.0 * dot


def _knn16_body(qp_ref, rt_ref, idx_ref, *, m_total):
    d = _sqdist(qp_ref, rt_ref)
    iota = lax.broadcasted_iota(jnp.int32, d.shape, 1)
    sel = lax.broadcasted_iota(jnp.int32, idx_ref.shape, 1)
    big = jnp.float32(jnp.inf)
    acc = jnp.zeros(idx_ref.shape, jnp.int32)
    for t in range(16):
        m = jnp.min(d, axis=1, keepdims=True)
        cand = jnp.where(d == m, iota, m_total)
        a = jnp.min(cand, axis=1, keepdims=True)
        acc = jnp.where(sel == t, a, acc)
        d = jnp.where(cand == a, big, d)
    idx_ref[...] = acc


def _run_knn16(qp, rt, qb):
    q_n = qp.shape[0]
    m = rt.shape[1]
    return pl.pallas_call(
        functools.partial(_knn16_body, m_total=m),
        grid=(q_n // qb,),
        in_specs=[pl.BlockSpec((qb, 8), lambda i: (i, 0)),
                  pl.BlockSpec((8, m), lambda i: (0, 0))],
        out_specs=pl.BlockSpec((qb, 16), lambda i: (i, 0)),
        out_shape=jax.ShapeDtypeStruct((q_n, 16), jnp.int32),
        compiler_params=pltpu.CompilerParams(
            dimension_semantics=("parallel",)),
    )(qp, rt)


def _knn3_body(qp_ref, rt_ref, idx_ref, w_ref, *, m_total):
    d = _sqdist(qp_ref, rt_ref)
    iota = lax.broadcasted_iota(jnp.int32, d.shape, 1)
    sel = lax.broadcasted_iota(jnp.int32, idx_ref.shape, 1)
    big = jnp.float32(jnp.inf)
    acc = jnp.zeros(idx_ref.shape, jnp.int32)
    wacc = jnp.zeros(w_ref.shape, jnp.float32)
    for t in range(3):
        m = jnp.min(d, axis=1, keepdims=True)
        cand = jnp.where(d == m, iota, m_total)
        a = jnp.min(cand, axis=1, keepdims=True)
        acc = jnp.where(sel == t, a, acc)
        wacc = jnp.where(sel == t, 1.0 / (m + 1e-8), wacc)
        d = jnp.where(cand == a, big, d)
    wsum = (wacc[:, 0:1] + wacc[:, 2:3]) + wacc[:, 1:2]
    wacc = wacc / wsum
    idx_ref[...] = acc
    w_ref[...] = wacc


def _run_knn3(qp, rt, qb):
    q_n = qp.shape[0]
    m = rt.shape[1]
    return pl.pallas_call(
        functools.partial(_knn3_body, m_total=m),
        grid=(q_n // qb,),
        in_specs=[pl.BlockSpec((qb, 8), lambda i: (i, 0)),
                  pl.BlockSpec((8, m), lambda i: (0, 0))],
        out_specs=[pl.BlockSpec((qb, 3), lambda i: (i, 0)),
                   pl.BlockSpec((qb, 3), lambda i: (i, 0))],
        out_shape=[jax.ShapeDtypeStruct((q_n, 3), jnp.int32),
                   jax.ShapeDtypeStruct((q_n, 3), jnp.float32)],
        compiler_params=pltpu.CompilerParams(
            dimension_semantics=("parallel",)),
    )(qp, rt)


_NC = 2
_NS = 16
_NW = _NC * _NS


def _sc_gather(table, idx, chunk):
    b_n = idx.shape[0]
    d = table.shape[1]
    b_per_w = b_n // _NW
    nch = b_per_w // chunk
    assert b_per_w % chunk == 0 and b_n % (8 * _NW) == 0 and d % 128 == 0
    mesh = plsc.VectorSubcoreMesh(core_axis_name="c", subcore_axis_name="s")

    @functools.partial(
        pl.kernel, mesh=mesh,
        out_type=jax.ShapeDtypeStruct((b_n, d), jnp.float32),
        scratch_types=[pltpu.VMEM((b_per_w,), jnp.int32),
                       pltpu.VMEM((chunk, d), jnp.float32),
                       pltpu.SemaphoreType.DMA],
    )
    def k(table_hbm, idx_hbm, out_hbm, idx_v, buf, sem):
        wid = lax.axis_index("s") * _NC + lax.axis_index("c")
        base = wid * b_per_w
        pltpu.sync_copy(idx_hbm.at[pl.ds(base, b_per_w)], idx_v)
        for c in range(nch):
            pltpu.async_copy(
                table_hbm.at[idx_v.at[pl.ds(c * chunk, chunk)]], buf, sem
            ).wait()
            pltpu.sync_copy(buf, out_hbm.at[pl.ds(base + c * chunk, chunk)])

    return k(table, idx)


def _bdot(x, w_ref):
    return lax.dot(x.astype(jnp.bfloat16), w_ref[...],
                   preferred_element_type=jnp.float32)


def _sa_mlp_body(g_ref, cor_ref, w0_ref, b0_ref, w1_ref, b1_ref,
                 w2_ref, b2_ref, out_ref, *, qb):
    g = g_ref[...]
    dpad = g.shape[1]
    cor = cor_ref[...]
    cor16 = jnp.broadcast_to(cor[:, None, :], (qb, 16, dpad)).reshape(qb * 16, dpad)
    h0 = jnp.maximum(_bdot(g - cor16, w0_ref) + b0_ref[...], 0.0)
    h1 = jnp.maximum(_bdot(h0, w1_ref) + b1_ref[...], 0.0)
    h2 = jnp.maximum(_bdot(h1, w2_ref) + b2_ref[...], 0.0)
    out_ref[...] = jnp.max(h2.reshape(qb, 16, h2.shape[1]), axis=1)


def _sa_mlp(g, cor, w0p, b0, w1, b1, w2, b2, qb):
    q_n = cor.shape[0]
    dpad = g.shape[1]
    c_out = w2.shape[1]
    c1 = w0p.shape[1]
    c2 = w1.shape[1]
    full = lambda shape: pl.BlockSpec(shape, lambda i: tuple(0 for _ in shape))
    return pl.pallas_call(
        functools.partial(_sa_mlp_body, qb=qb),
        grid=(q_n // qb,),
        in_specs=[pl.BlockSpec((qb * 16, dpad), lambda i: (i, 0)),
                  pl.BlockSpec((qb, dpad), lambda i: (i, 0)),
                  full((dpad, c1)), full((1, c1)),
                  full((c1, c2)), full((1, c2)),
                  full((c2, c_out)), full((1, c_out))],
        out_specs=pl.BlockSpec((qb, c_out), lambda i: (i, 0)),
        out_shape=jax.ShapeDtypeStruct((q_n, c_out), jnp.float32),
        compiler_params=pltpu.CompilerParams(
            dimension_semantics=("parallel",)),
    )(g, cor, w0p, b0, w1, b1, w2, b2)


def _fp_body(gr_ref, w0c_ref, w1c_ref, w2c_ref, skip_ref, wi_ref, ws_ref,
             b0_ref, w1_ref, b1_ref, out_ref):
    x = gr_ref[...]
    c = wi_ref.shape[0]
    interp = (x[:, :c] * w0c_ref[...] + x[:, c:2 * c] * w1c_ref[...]
              + x[:, 2 * c:] * w2c_ref[...])
    h0 = (_bdot(interp, wi_ref) + _bdot(skip_ref[...], ws_ref) + b0_ref[...])
    h0 = jnp.maximum(h0, 0.0)
    h1 = jnp.maximum(_bdot(h0, w1_ref) + b1_ref[...], 0.0)
    out_ref[...] = h1


def _fp_mlp(gr, w3, skip, wi, ws, b0, w1, b1, qb):
    q_n = gr.shape[0]
    c3 = gr.shape[1]
    c = wi.shape[0]
    cs = ws.shape[0]
    c1 = wi.shape[1]
    c2 = w1.shape[1]
    w0c = w3[:, 0:1]
    w1c = w3[:, 1:2]
    w2c = w3[:, 2:3]
    full = lambda shape: pl.BlockSpec(shape, lambda i: tuple(0 for _ in shape))
    return pl.pallas_call(
        _fp_body,
        grid=(q_n // qb,),
        in_specs=[pl.BlockSpec((qb, c3), lambda i: (i, 0)),
                  pl.BlockSpec((qb, 1), lambda i: (i, 0)),
                  pl.BlockSpec((qb, 1), lambda i: (i, 0)),
                  pl.BlockSpec((qb, 1), lambda i: (i, 0)),
                  pl.BlockSpec((qb, cs), lambda i: (i, 0)),
                  full((c, c1)), full((cs, c1)), full((1, c1)),
                  full((c1, c2)), full((1, c2))],
        out_specs=pl.BlockSpec((qb, c2), lambda i: (i, 0)),
        out_shape=jax.ShapeDtypeStruct((q_n, c2), jnp.float32),
        compiler_params=pltpu.CompilerParams(
            dimension_semantics=("parallel",)),
    )(gr, w0c, w1c, w2c, skip, wi, ws, b0, w1, b1)


def _pad_cols(x, d):
    return jnp.pad(x, ((0, 0), (0, d - x.shape[1])))


def kernel(point_bxyz, point_feat, sa1_W0, sa1_b0, sa1_W1, sa1_b1, sa1_W2,
           sa1_b2, sa2_W0, sa2_b0, sa2_W1, sa2_b1, sa2_W2, sa2_b2, fp1_W0,
           fp1_b0, fp1_W1, fp1_b1, fp2_W0, fp2_b0, fp2_W1, fp2_b1):
    xyz0 = point_bxyz[:, 1:4]
    qp0 = _pad_cols(xyz0, 8)
    qp1 = qp0[::4]
    qp2 = qp1[::4]
    rt0 = qp0.T
    rt1 = qp1.T
    rt2 = qp2.T

    row = lambda b: b.reshape(1, -1)
    bf = lambda w: w.astype(jnp.bfloat16)
    xyz1 = xyz0[::4]
    xyz2 = xyz1[::4]

    nn1 = _run_knn16(qp1, rt0, qb=128)
    table1 = _pad_cols(jnp.concatenate([point_feat, xyz0], axis=1), 128)
    g1 = _sc_gather(table1, nn1.reshape(-1), chunk=512)
    w0p = _pad_cols(sa1_W0.T, 128).T
    cor1 = _pad_cols(jnp.concatenate([jnp.zeros((4096, 64), jnp.float32),
                                      xyz1], axis=1), 128)
    f1 = _sa_mlp(g1, cor1, bf(w0p), row(sa1_b0), bf(sa1_W1), row(sa1_b1),
                 bf(sa1_W2), row(sa1_b2), qb=512)

    nn2 = _run_knn16(qp2, rt1, qb=256)
    table2 = _pad_cols(jnp.concatenate([f1, xyz1], axis=1), 256)
    g2 = _sc_gather(table2, nn2.reshape(-1), chunk=256)
    w0p2 = _pad_cols(sa2_W0.T, 256).T
    cor2 = _pad_cols(jnp.concatenate([jnp.zeros((1024, 128), jnp.float32),
                                      xyz2], axis=1), 256)
    f2 = _sa_mlp(g2, cor2, bf(w0p2), row(sa2_b0), bf(sa2_W1), row(sa2_b1),
                 bf(sa2_W2), row(sa2_b2), qb=512)

    nn3a, w3a = _run_knn3(qp1, rt2, qb=512)
    gf1 = _sc_gather(f2, nn3a.reshape(-1), chunk=384)
    f1p = _fp_mlp(gf1.reshape(4096, 768), w3a, f1, bf(fp1_W0[:256]),
                  bf(fp1_W0[256:]), row(fp1_b0), bf(fp1_W1), row(fp1_b1),
                  qb=512)

    nn3b, w3b = _run_knn3(qp0, rt1, qb=512)
    gf2 = _sc_gather(f1p, nn3b.reshape(-1), chunk=384)
    out = _fp_mlp(gf2.reshape(16384, 768), w3b, point_feat, bf(fp2_W0[:256]),
                  bf(fp2_W0[256:]), row(fp2_b0), bf(fp2_W1), row(fp2_b1),
                  qb=1024)
    return out
